# baseline (speedup 1.0000x reference)
import functools

import jax
import jax.numpy as jnp
from jax import lax
from jax.experimental import pallas as pl
from jax.experimental.pallas import tpu as pltpu
from jax.experimental.pallas import tpu_sc as plsc

_BATCH = 4
_NUM_CORES = 2
_NUM_SUBCORES = 16
_NUM_WORKERS = _NUM_CORES * _NUM_SUBCORES
_CHUNK = 128


def kernel(x, pe_weight):
    batch = x.shape[0]
    max_len, d_model = pe_weight.shape
    assert batch == _BATCH and max_len % _NUM_WORKERS == 0
    rows_per_worker = max_len // _NUM_WORKERS
    assert rows_per_worker % _CHUNK == 0
    n_chunks = rows_per_worker // _CHUNK

    mesh = plsc.VectorSubcoreMesh(core_axis_name="c", subcore_axis_name="s")

    @functools.partial(
        pl.kernel,
        mesh=mesh,
        out_type=jax.ShapeDtypeStruct((batch, max_len, d_model), pe_weight.dtype),
        scratch_types=[
            pltpu.VMEM((_CHUNK, d_model), pe_weight.dtype),
            pltpu.SemaphoreType.DMA,
        ],
        compiler_params=pltpu.CompilerParams(use_tc_tiling_on_sc=True),
    )
    def _sc_bcast(pe_hbm, out_hbm, buf, sem):
        wid = lax.axis_index("s") * _NUM_CORES + lax.axis_index("c")
        base = wid * rows_per_worker

        def body(i, _):
            row = base + i * _CHUNK
            pltpu.sync_copy(pe_hbm.at[pl.ds(row, _CHUNK)], buf)
            copies = [
                pltpu.make_async_copy(
                    buf, out_hbm.at[b, pl.ds(row, _CHUNK)], sem
                )
                for b in range(_BATCH)
            ]
            for c in copies:
                c.start()
            for c in copies:
                c.wait()
            return ()

        lax.fori_loop(0, n_chunks, body, ())

    return _sc_bcast(pe_weight)


# speedup vs baseline: 1.0052x; 1.0052x over previous
import functools

import jax
import jax.numpy as jnp
from jax import lax
from jax.experimental import pallas as pl
from jax.experimental.pallas import tpu as pltpu
from jax.experimental.pallas import tpu_sc as plsc

_BATCH = 4
_NUM_CORES = 2
_NUM_SUBCORES = 16
_NUM_WORKERS = _NUM_CORES * _NUM_SUBCORES
_CHUNK = 128


def kernel(x, pe_weight):
    batch = x.shape[0]
    max_len, d_model = pe_weight.shape
    assert batch == _BATCH and max_len % _NUM_WORKERS == 0
    rows_per_worker = max_len // _NUM_WORKERS
    assert rows_per_worker % _CHUNK == 0
    n_chunks = rows_per_worker // _CHUNK

    mesh = plsc.VectorSubcoreMesh(core_axis_name="c", subcore_axis_name="s")

    @functools.partial(
        pl.kernel,
        mesh=mesh,
        out_type=jax.ShapeDtypeStruct((batch, max_len, d_model), pe_weight.dtype),
        scratch_types=[
            pltpu.VMEM((_CHUNK, d_model), pe_weight.dtype),
            pltpu.SemaphoreType.DMA,
        ],
    )
    def _sc_bcast(pe_hbm, out_hbm, buf, sem):
        wid = lax.axis_index("s") * _NUM_CORES + lax.axis_index("c")
        base = wid * rows_per_worker

        def body(i, _):
            row = base + i * _CHUNK
            pltpu.sync_copy(pe_hbm.at[pl.ds(row, _CHUNK)], buf)
            copies = [
                pltpu.make_async_copy(
                    buf, out_hbm.at[b, pl.ds(row, _CHUNK)], sem
                )
                for b in range(_BATCH)
            ]
            for c in copies:
                c.start()
            for c in copies:
                c.wait()
            return ()

        lax.fori_loop(0, n_chunks, body, ())

    return _sc_bcast(pe_weight)
